# baseline (speedup 1.0000x reference)
import functools

import jax
import jax.numpy as jnp
from jax import lax
from jax.experimental import pallas as pl
from jax.experimental.pallas import tpu as pltpu
from jax.experimental.pallas import tpu_sc as plsc

_LANES = 16
_NBUF = 4


def _make_sc_kernel(n_nodes, d_feat, n_edges, n_cores, n_subcores, chunk):
    n_workers = n_cores * n_subcores
    per_worker = n_edges // n_workers
    n_chunks = per_worker // chunk
    n_rounds = n_chunks // _NBUF
    n_tail = n_chunks % _NBUF
    n_groups = chunk // _LANES
    n_words = d_feat // 2

    mesh = plsc.VectorSubcoreMesh(core_axis_name="c", subcore_axis_name="s")

    @functools.partial(
        pl.kernel,
        mesh=mesh,
        compiler_params=pltpu.CompilerParams(needs_layout_passes=False,
                                             use_tc_tiling_on_sc=False),
        out_type=jax.ShapeDtypeStruct((n_workers * n_chunks, chunk),
                                      jnp.float32),
        scratch_types=[
            pltpu.VMEM((2, n_chunks, chunk), jnp.int32),
            pltpu.VMEM((_NBUF, chunk, n_words), jnp.int32),
            pltpu.VMEM((_NBUF, chunk, n_words), jnp.int32),
            pltpu.VMEM((_NBUF, chunk), jnp.float32),
            pltpu.VMEM((_LANES, _LANES), jnp.float32),
            pltpu.SemaphoreType.DMA((_NBUF,)),
            pltpu.SemaphoreType.DMA((_NBUF,)),
        ],
    )
    def sc_kernel(h_hbm, eidx_hbm, out_hbm, eidx, ubuf, vbuf, obuf, pscr,
                  gsem, osem):
        wid = lax.axis_index("s") * n_cores + lax.axis_index("c")
        pltpu.sync_copy(eidx_hbm.at[0, wid], eidx.at[0])
        pltpu.sync_copy(eidx_hbm.at[1, wid], eidx.at[1])
        lane = lax.iota(jnp.int32, _LANES)

        def launch_gather(ci, b):
            pltpu.async_copy(h_hbm.at[eidx.at[0, ci]], ubuf.at[b],
                             gsem.at[b])
            pltpu.async_copy(h_hbm.at[eidx.at[1, ci]], vbuf.at[b],
                             gsem.at[b])

        def wait_gather(ci, b):
            pltpu.make_async_copy(h_hbm.at[eidx.at[0, ci]], ubuf.at[b],
                                  gsem.at[b]).wait()
            pltpu.make_async_copy(h_hbm.at[eidx.at[1, ci]], vbuf.at[b],
                                  gsem.at[b]).wait()

        def compute(ci, b):
            def group_body(g, _):
                for e in range(_LANES):
                    j = g * _LANES + e
                    acc = jnp.zeros((_LANES,), jnp.float32)
                    for k in range(n_words // (2 * _LANES)):
                        p0 = (plsc.bitcast(
                            ubuf[b, j, pl.ds(2 * k * _LANES, _LANES)],
                            jnp.bfloat16)
                            * plsc.bitcast(
                            vbuf[b, j, pl.ds(2 * k * _LANES, _LANES)],
                            jnp.bfloat16))
                        p1 = (plsc.bitcast(
                            ubuf[b, j, pl.ds((2 * k + 1) * _LANES, _LANES)],
                            jnp.bfloat16)
                            * plsc.bitcast(
                            vbuf[b, j, pl.ds((2 * k + 1) * _LANES, _LANES)],
                            jnp.bfloat16))
                        pa, pb = plsc.unpack(
                            p0 + p1, format=plsc.PackFormat.INTERLEAVED)
                        acc = acc + pa + pb
                    pscr[e, :] = acc
                accs = jnp.zeros((_LANES,), jnp.float32)
                for k in range(_LANES):
                    col = jnp.bitwise_and(lane + k, _LANES - 1)
                    accs = accs + plsc.load_gather(pscr, [lane, col])
                obuf[b, pl.ds(g * _LANES, _LANES)] = accs
                return 0

            lax.fori_loop(0, n_groups, group_body, 0)
            pltpu.async_copy(obuf.at[b], out_hbm.at[wid * n_chunks + ci],
                             osem.at[b])

        def wait_out(ci, b):
            pltpu.make_async_copy(obuf.at[b],
                                  out_hbm.at[wid * n_chunks + ci],
                                  osem.at[b]).wait()

        for b in range(_NBUF):
            launch_gather(b, b)

        def round_body(p, _):
            ci0 = p * _NBUF
            for b in range(_NBUF):
                ci = ci0 + b
                wait_gather(ci, b)

                @pl.when(ci >= _NBUF)
                def _():
                    wait_out(ci - _NBUF, b)

                compute(ci, b)

                @pl.when(ci + _NBUF < n_chunks)
                def _():
                    launch_gather(ci + _NBUF, b)

            return 0

        lax.fori_loop(0, n_rounds, round_body, 0)
        for t in range(n_tail):
            ci = n_rounds * _NBUF + t
            wait_gather(ci, t)
            wait_out(ci - _NBUF, t)
            compute(ci, t)
        for b in range(_NBUF):
            ci = n_chunks - _NBUF + b
            wait_out(ci, ci % _NBUF)

    return sc_kernel


def kernel(h, edge_index):
    n_nodes, d_feat = h.shape
    n_edges = edge_index.shape[1]
    chunk = 80
    info = plsc.get_sparse_core_info()
    n_workers = info.num_cores * info.num_subcores
    n_chunks = (n_edges // n_workers) // chunk
    eidx = edge_index.astype(jnp.int32).reshape(2, n_workers, n_chunks, chunk)
    h32 = jax.lax.bitcast_convert_type(
        h.astype(jnp.bfloat16).reshape(n_nodes, d_feat // 2, 2), jnp.int32)
    sc_fn = _make_sc_kernel(n_nodes, d_feat, n_edges,
                            info.num_cores, info.num_subcores, chunk)
    out = sc_fn(h32, eidx)
    return out.reshape(n_edges, 1)


# speedup vs baseline: 1.0239x; 1.0239x over previous
import functools

import jax
import jax.numpy as jnp
from jax import lax
from jax.experimental import pallas as pl
from jax.experimental.pallas import tpu as pltpu
from jax.experimental.pallas import tpu_sc as plsc

_LANES = 16


def _make_sc_kernel(n_nodes, d_feat, n_edges, n_cores, n_subcores, chunk):
    n_workers = n_cores * n_subcores
    per_worker = n_edges // n_workers
    n_chunks = per_worker // chunk
    n_pairs = n_chunks // 2
    n_groups = chunk // _LANES
    n_words = d_feat // 2

    mesh = plsc.VectorSubcoreMesh(core_axis_name="c", subcore_axis_name="s")

    @functools.partial(
        pl.kernel,
        mesh=mesh,
        compiler_params=pltpu.CompilerParams(needs_layout_passes=False,
                                             use_tc_tiling_on_sc=False),
        out_type=jax.ShapeDtypeStruct((n_workers * n_chunks, chunk),
                                      jnp.float32),
        scratch_types=[
            pltpu.VMEM((2, n_chunks, chunk), jnp.int32),
            pltpu.VMEM((2, chunk, n_words), jnp.int32),
            pltpu.VMEM((2, chunk, n_words), jnp.int32),
            pltpu.VMEM((2, chunk), jnp.float32),
            pltpu.VMEM((_LANES, _LANES), jnp.float32),
            pltpu.SemaphoreType.DMA((2,)),
            pltpu.SemaphoreType.DMA((2,)),
        ],
    )
    def sc_kernel(h_hbm, eidx_hbm, out_hbm, eidx, ubuf, vbuf, obuf, pscr,
                  gsem, osem):
        wid = lax.axis_index("s") * n_cores + lax.axis_index("c")
        pltpu.sync_copy(eidx_hbm.at[0, wid], eidx.at[0])
        pltpu.sync_copy(eidx_hbm.at[1, wid], eidx.at[1])
        lane = lax.iota(jnp.int32, _LANES)

        def launch_gather(ci, b):
            pltpu.async_copy(h_hbm.at[eidx.at[0, ci]], ubuf.at[b],
                             gsem.at[b])
            pltpu.async_copy(h_hbm.at[eidx.at[1, ci]], vbuf.at[b],
                             gsem.at[b])

        def wait_gather(ci, b):
            pltpu.make_async_copy(h_hbm.at[eidx.at[0, ci]], ubuf.at[b],
                                  gsem.at[b]).wait()
            pltpu.make_async_copy(h_hbm.at[eidx.at[1, ci]], vbuf.at[b],
                                  gsem.at[b]).wait()

        def compute(ci, b):
            def group_body(g, _):
                for e in range(_LANES):
                    j = g * _LANES + e
                    acc = jnp.zeros((_LANES,), jnp.float32)
                    for k in range(n_words // (2 * _LANES)):
                        p0 = (plsc.bitcast(
                            ubuf[b, j, pl.ds(2 * k * _LANES, _LANES)],
                            jnp.bfloat16)
                            * plsc.bitcast(
                            vbuf[b, j, pl.ds(2 * k * _LANES, _LANES)],
                            jnp.bfloat16))
                        p1 = (plsc.bitcast(
                            ubuf[b, j, pl.ds((2 * k + 1) * _LANES, _LANES)],
                            jnp.bfloat16)
                            * plsc.bitcast(
                            vbuf[b, j, pl.ds((2 * k + 1) * _LANES, _LANES)],
                            jnp.bfloat16))
                        pa, pb = plsc.unpack(
                            p0 + p1, format=plsc.PackFormat.INTERLEAVED)
                        acc = acc + pa + pb
                    pscr[e, :] = acc
                parts = [jnp.zeros((_LANES,), jnp.float32) for _ in range(4)]
                for k in range(_LANES):
                    col = jnp.bitwise_and(lane + k, _LANES - 1)
                    parts[k % 4] = parts[k % 4] + plsc.load_gather(
                        pscr, [lane, col])
                obuf[b, pl.ds(g * _LANES, _LANES)] = (
                    (parts[0] + parts[1]) + (parts[2] + parts[3]))
                return 0

            lax.fori_loop(0, n_groups, group_body, 0)
            pltpu.async_copy(obuf.at[b], out_hbm.at[wid * n_chunks + ci],
                             osem.at[b])

        def wait_out(ci, b):
            pltpu.make_async_copy(obuf.at[b],
                                  out_hbm.at[wid * n_chunks + ci],
                                  osem.at[b]).wait()

        launch_gather(0, 0)

        def pair_body(p, _):
            ci0 = 2 * p
            launch_gather(ci0 + 1, 1)
            wait_gather(ci0, 0)

            @pl.when(p > 0)
            def _():
                wait_out(ci0 - 2, 0)

            compute(ci0, 0)

            @pl.when(ci0 + 2 < n_chunks)
            def _():
                launch_gather(ci0 + 2, 0)

            wait_gather(ci0 + 1, 1)

            @pl.when(p > 0)
            def _():
                wait_out(ci0 - 1, 1)

            compute(ci0 + 1, 1)
            return 0

        lax.fori_loop(0, n_pairs, pair_body, 0)
        if n_chunks % 2:
            tail = n_chunks - 1
            wait_gather(tail, 0)
            wait_out(tail - 2, 0)
            compute(tail, 0)
            wait_out(tail - 1, 1)
            wait_out(tail, 0)
        else:
            wait_out(n_chunks - 2, 0)
            wait_out(n_chunks - 1, 1)

    return sc_kernel


def kernel(h, edge_index):
    n_nodes, d_feat = h.shape
    n_edges = edge_index.shape[1]
    chunk = 80
    info = plsc.get_sparse_core_info()
    n_workers = info.num_cores * info.num_subcores
    n_chunks = (n_edges // n_workers) // chunk
    eidx = edge_index.astype(jnp.int32).reshape(2, n_workers, n_chunks, chunk)
    h32 = jax.lax.bitcast_convert_type(
        h.astype(jnp.bfloat16).reshape(n_nodes, d_feat // 2, 2), jnp.int32)
    sc_fn = _make_sc_kernel(n_nodes, d_feat, n_edges,
                            info.num_cores, info.num_subcores, chunk)
    out = sc_fn(h32, eidx)
    return out.reshape(n_edges, 1)


# speedup vs baseline: 1.0253x; 1.0013x over previous
import functools

import jax
import jax.numpy as jnp
from jax import lax
from jax.experimental import pallas as pl
from jax.experimental.pallas import tpu as pltpu
from jax.experimental.pallas import tpu_sc as plsc

_LANES = 16


def _make_sc_kernel(n_nodes, d_feat, n_edges, n_cores, n_subcores, chunk):
    n_workers = n_cores * n_subcores
    per_worker = n_edges // n_workers
    n_chunks = per_worker // chunk
    n_pairs = n_chunks // 2
    n_groups = chunk // _LANES
    n_words = d_feat // 2

    mesh = plsc.VectorSubcoreMesh(core_axis_name="c", subcore_axis_name="s")

    @functools.partial(
        pl.kernel,
        mesh=mesh,
        compiler_params=pltpu.CompilerParams(needs_layout_passes=False,
                                             use_tc_tiling_on_sc=False),
        out_type=jax.ShapeDtypeStruct((n_workers * n_chunks, chunk),
                                      jnp.float32),
        scratch_types=[
            pltpu.VMEM((2, n_chunks, chunk), jnp.int32),
            pltpu.VMEM((2, chunk, n_words), jnp.int32),
            pltpu.VMEM((2, chunk, n_words), jnp.int32),
            pltpu.VMEM((2, chunk), jnp.float32),
            pltpu.VMEM((_LANES, _LANES), jnp.float32),
            pltpu.SemaphoreType.DMA((2,)),
            pltpu.SemaphoreType.DMA((2,)),
        ],
    )
    def sc_kernel(h_hbm, eidx_hbm, out_hbm, eidx, ubuf, vbuf, obuf, pscr,
                  gsem, osem):
        wid = lax.axis_index("s") * n_cores + lax.axis_index("c")
        pltpu.sync_copy(eidx_hbm.at[0, wid], eidx.at[0])
        pltpu.sync_copy(eidx_hbm.at[1, wid], eidx.at[1])
        lane = lax.iota(jnp.int32, _LANES)

        def launch_gather(ci, b):
            pltpu.async_copy(h_hbm.at[eidx.at[0, ci]], ubuf.at[b],
                             gsem.at[b])
            pltpu.async_copy(h_hbm.at[eidx.at[1, ci]], vbuf.at[b],
                             gsem.at[b])

        def wait_gather(ci, b):
            pltpu.make_async_copy(h_hbm.at[eidx.at[0, ci]], ubuf.at[b],
                                  gsem.at[b]).wait()
            pltpu.make_async_copy(h_hbm.at[eidx.at[1, ci]], vbuf.at[b],
                                  gsem.at[b]).wait()

        def compute(ci, b):
            def group_body(g, _):
                for e in range(_LANES):
                    j = g * _LANES + e
                    halves = []
                    for k in range(n_words // (2 * _LANES)):
                        p0 = (plsc.bitcast(
                            ubuf[b, j, pl.ds(2 * k * _LANES, _LANES)],
                            jnp.bfloat16)
                            * plsc.bitcast(
                            vbuf[b, j, pl.ds(2 * k * _LANES, _LANES)],
                            jnp.bfloat16))
                        p1 = (plsc.bitcast(
                            ubuf[b, j, pl.ds((2 * k + 1) * _LANES, _LANES)],
                            jnp.bfloat16)
                            * plsc.bitcast(
                            vbuf[b, j, pl.ds((2 * k + 1) * _LANES, _LANES)],
                            jnp.bfloat16))
                        pa, pb = plsc.unpack(
                            p0 + p1, format=plsc.PackFormat.INTERLEAVED)
                        halves.append(pa + pb)
                    acc = halves[0]
                    for hh in halves[1:]:
                        acc = acc + hh
                    pscr[e, :] = acc
                parts = [jnp.zeros((_LANES,), jnp.float32) for _ in range(4)]
                for k in range(_LANES):
                    col = jnp.bitwise_and(lane + k, _LANES - 1)
                    parts[k % 4] = parts[k % 4] + plsc.load_gather(
                        pscr, [lane, col])
                obuf[b, pl.ds(g * _LANES, _LANES)] = (
                    (parts[0] + parts[1]) + (parts[2] + parts[3]))
                return 0

            lax.fori_loop(0, n_groups, group_body, 0)
            pltpu.async_copy(obuf.at[b], out_hbm.at[wid * n_chunks + ci],
                             osem.at[b])

        def wait_out(ci, b):
            pltpu.make_async_copy(obuf.at[b],
                                  out_hbm.at[wid * n_chunks + ci],
                                  osem.at[b]).wait()

        launch_gather(0, 0)

        def pair_body(p, _):
            ci0 = 2 * p
            launch_gather(ci0 + 1, 1)
            wait_gather(ci0, 0)

            @pl.when(p > 0)
            def _():
                wait_out(ci0 - 2, 0)

            compute(ci0, 0)

            @pl.when(ci0 + 2 < n_chunks)
            def _():
                launch_gather(ci0 + 2, 0)

            wait_gather(ci0 + 1, 1)

            @pl.when(p > 0)
            def _():
                wait_out(ci0 - 1, 1)

            compute(ci0 + 1, 1)
            return 0

        lax.fori_loop(0, n_pairs, pair_body, 0)
        if n_chunks % 2:
            tail = n_chunks - 1
            wait_gather(tail, 0)
            wait_out(tail - 2, 0)
            compute(tail, 0)
            wait_out(tail - 1, 1)
            wait_out(tail, 0)
        else:
            wait_out(n_chunks - 2, 0)
            wait_out(n_chunks - 1, 1)

    return sc_kernel


def kernel(h, edge_index):
    n_nodes, d_feat = h.shape
    n_edges = edge_index.shape[1]
    chunk = 80
    info = plsc.get_sparse_core_info()
    n_workers = info.num_cores * info.num_subcores
    n_chunks = (n_edges // n_workers) // chunk
    eidx = edge_index.astype(jnp.int32).reshape(2, n_workers, n_chunks, chunk)
    h32 = jax.lax.bitcast_convert_type(
        h.astype(jnp.bfloat16).reshape(n_nodes, d_feat // 2, 2), jnp.int32)
    sc_fn = _make_sc_kernel(n_nodes, d_feat, n_edges,
                            info.num_cores, info.num_subcores, chunk)
    out = sc_fn(h32, eidx)
    return out.reshape(n_edges, 1)
